# baseline (speedup 1.0000x reference)
import jax
import jax.numpy as jnp
from jax import lax
from jax.experimental import pallas as pl
from jax.experimental.pallas import tpu as pltpu
from jax.experimental.pallas import tpu_sc as plsc

VOCAB = 1000000
D_MODEL = 64
BATCH = 4096
SEQ = 200

NC = 2
NS = 16
NW = NC * NS
BBLK = BATCH // NW
JH = D_MODEL // 8
LANES = 16
WPAD = 133
PROW = 2 * D_MODEL

NBLK_FULL = VOCAB // 128
TAIL_START = NBLK_FULL * 128
NBLK_PW = 246
NAB = 3

NG = 4
NWR = 4


def _detile_kernel(tt_hbm, out_hbm, vbufs, wbufs, gsems, wsems):
    wid = lax.axis_index("s") * NC + lax.axis_index("c")
    base = wid * NBLK_PW

    lanes = lax.iota(jnp.int32, LANES)
    il_c = [k * LANES + lanes for k in range(128 // LANES)]

    def blk_start(t):
        return lax.rem(base + t, NBLK_FULL) * 128

    def transpose_block(v, w):
        def jbody(j, carry):
            jidx = jnp.full((LANES,), j, jnp.int32)
            vecs = [v[j, pl.ds(k * LANES, LANES)] for k in range(128 // LANES)]
            for k, vec in enumerate(vecs):
                plsc.store_scatter(w, [il_c[k], jidx], vec)
            return carry

        lax.fori_loop(0, D_MODEL, jbody, 0)

    def fetch_slab(start, vb, sem):
        for jh in range(JH):
            pltpu.async_copy(
                tt_hbm.at[pl.ds(jh * 8, 8), pl.ds(start, 128)],
                vb.at[pl.ds(jh * 8, 8), :],
                sem,
            )

    def wait_slab(start, vb, sem):
        for jh in range(JH):
            pltpu.make_async_copy(
                tt_hbm.at[pl.ds(jh * 8, 8), pl.ds(start, 128)],
                vb.at[pl.ds(jh * 8, 8), :],
                sem,
            ).wait()

    for a in range(NAB):
        fetch_slab(blk_start(a), vbufs[a], gsems.at[a])

    def group_body(grp, carry):
        for slot in range(NAB):
            t = grp * NAB + slot
            start = blk_start(t)
            wait_slab(start, vbufs[slot], gsems.at[slot])

            @pl.when(t >= NAB)
            def _():
                pltpu.make_async_copy(
                    wbufs[slot].at[:, pl.ds(0, PROW)],
                    out_hbm.at[pl.ds(blk_start(t - NAB), 128)],
                    wsems.at[slot],
                ).wait()

            transpose_block(vbufs[slot], wbufs[slot])
            pltpu.async_copy(
                wbufs[slot].at[:, pl.ds(0, PROW)],
                out_hbm.at[pl.ds(start, 128)],
                wsems.at[slot],
            )

            nt = t + NAB

            @pl.when(nt < NBLK_PW)
            def _():
                fetch_slab(blk_start(nt), vbufs[slot], gsems.at[slot])

        return carry

    lax.fori_loop(0, NBLK_PW // NAB, group_body, 0)

    for slot in range(NAB):
        t = NBLK_PW - NAB + slot
        pltpu.make_async_copy(
            wbufs[slot].at[:, pl.ds(0, PROW)],
            out_hbm.at[pl.ds(blk_start(t), 128)],
            wsems.at[slot],
        ).wait()


def _embed_kernel(xt_hbm, table_hbm, out_hbm, idx_v, gbufs, wbufs, gsems, wsems):
    wid = lax.axis_index("s") * NC + lax.axis_index("c")

    pltpu.sync_copy(xt_hbm.at[:, pl.ds(wid * BBLK, BBLK)], idx_v)

    lanes = lax.iota(jnp.int32, LANES)

    for b in range(NG):
        pltpu.async_copy(table_hbm.at[idx_v.at[b]], gbufs[b], gsems.at[b])

    jh_c = [(k * LANES + lanes) // 8 for k in range(D_MODEL // LANES)]
    jl_c = [(k * LANES + lanes) % 8 for k in range(D_MODEL // LANES)]

    def transpose_chunk(g, w):
        def bbody(b2, carry):
            for db in range(2):
                b = b2 * 2 + db
                bidx = jnp.full((LANES,), b, jnp.int32)
                vecs = [
                    g[b, pl.ds(k * LANES, LANES)]
                    for k in range(D_MODEL // LANES)
                ]
                for k, vec in enumerate(vecs):
                    plsc.store_scatter(w, [jh_c[k], jl_c[k], bidx], vec)
            return carry

        lax.fori_loop(0, BBLK // 2, bbody, 0)

    def group_body(grp, carry):
        for b in range(NG):
            s = grp * NG + b
            wb = b % NWR
            pltpu.make_async_copy(
                table_hbm.at[idx_v.at[s]], gbufs[b], gsems.at[b]
            ).wait()

            @pl.when(s >= NWR)
            def _():
                pltpu.make_async_copy(
                    wbufs[wb].at[:, :, pl.ds(0, BBLK)],
                    out_hbm.at[s - NWR, :, wid],
                    wsems.at[wb],
                ).wait()

            transpose_chunk(gbufs[b], wbufs[wb])
            pltpu.async_copy(
                wbufs[wb].at[:, :, pl.ds(0, BBLK)],
                out_hbm.at[s, :, wid],
                wsems.at[wb],
            )

            nxt = s + NG

            @pl.when(nxt < SEQ)
            def _():
                pltpu.async_copy(
                    table_hbm.at[idx_v.at[nxt]], gbufs[b], gsems.at[b]
                )

        return carry

    lax.fori_loop(0, SEQ // NG, group_body, 0)

    for wb in range(NWR):
        s = SEQ - NWR + wb
        pltpu.make_async_copy(
            wbufs[wb].at[:, :, pl.ds(0, BBLK)], out_hbm.at[s, :, wid], wsems.at[wb]
        ).wait()


@jax.jit
def kernel(x, table):
    xt = x.T.astype(jnp.int32)
    tt = table.T
    mesh = plsc.VectorSubcoreMesh(core_axis_name="c", subcore_axis_name="s")

    detile = pl.kernel(
        _detile_kernel,
        out_type=jax.ShapeDtypeStruct((VOCAB, PROW), jnp.float32),
        mesh=mesh,
        scratch_types=[
            [pltpu.VMEM((D_MODEL, 128), jnp.float32) for _ in range(NAB)],
            [pltpu.VMEM((128, WPAD), jnp.float32) for _ in range(NAB)],
            pltpu.SemaphoreType.DMA((NAB,)),
            pltpu.SemaphoreType.DMA((NAB,)),
        ],
        compiler_params=pltpu.CompilerParams(
            use_tc_tiling_on_sc=True, needs_layout_passes=False
        ),
    )
    t128 = detile(tt)
    tail128 = jnp.pad(table[TAIL_START:], ((0, 0), (0, D_MODEL)))
    t128 = lax.dynamic_update_slice(t128, tail128, (TAIL_START, 0))

    run = pl.kernel(
        _embed_kernel,
        out_type=jax.ShapeDtypeStruct((SEQ, JH, NW, 8, BBLK), jnp.float32),
        mesh=mesh,
        scratch_types=[
            pltpu.VMEM((SEQ, BBLK), jnp.int32),
            [pltpu.VMEM((BBLK, PROW), jnp.float32) for _ in range(NG)],
            [pltpu.VMEM((JH, 8, WPAD), jnp.float32) for _ in range(NWR)],
            pltpu.SemaphoreType.DMA((NG,)),
            pltpu.SemaphoreType.DMA((NWR,)),
        ],
        compiler_params=pltpu.CompilerParams(
            use_tc_tiling_on_sc=False, needs_layout_passes=False
        ),
    )
    p = run(xt, t128)
    return p.transpose(2, 4, 0, 1, 3).reshape(BATCH, SEQ, D_MODEL)


# speedup vs baseline: 1.7141x; 1.7141x over previous
import jax
import jax.numpy as jnp
from jax import lax
from jax.experimental import pallas as pl
from jax.experimental.pallas import tpu as pltpu
from jax.experimental.pallas import tpu_sc as plsc

VOCAB = 1000000
D_MODEL = 64
BATCH = 4096
SEQ = 200

NC = 2
NS = 16
NW = NC * NS
BBLK = BATCH // NW
NG = 8
NWR = 4
JH = D_MODEL // 8
LANES = 16
WPAD = 133


def _embed_kernel(xt_hbm, table_hbm, out_hbm, idx_v, gbufs, wbufs, gsems, wsems):
    wid = lax.axis_index("s") * NC + lax.axis_index("c")

    pltpu.sync_copy(xt_hbm.at[:, pl.ds(wid * BBLK, BBLK)], idx_v)

    lanes = lax.iota(jnp.int32, LANES)

    for b in range(NG):
        pltpu.async_copy(table_hbm.at[idx_v.at[b]], gbufs[b], gsems.at[b])

    jh_c = [(k * LANES + lanes) // 8 for k in range(D_MODEL // LANES)]
    jl_c = [(k * LANES + lanes) % 8 for k in range(D_MODEL // LANES)]

    def transpose_chunk(g, w):
        def bbody(b4, carry):
            for db in range(4):
                b = b4 * 4 + db
                bidx = jnp.full((LANES,), b, jnp.int32)
                vecs = [
                    g[b, pl.ds(k * LANES, LANES)]
                    for k in range(D_MODEL // LANES)
                ]
                for k, vec in enumerate(vecs):
                    plsc.store_scatter(w, [jh_c[k], jl_c[k], bidx], vec)
            return carry

        lax.fori_loop(0, BBLK // 4, bbody, 0)

    def group_body(grp, carry):
        for b in range(NG):
            s = grp * NG + b
            wb = b % NWR
            pltpu.make_async_copy(
                table_hbm.at[idx_v.at[s]], gbufs[b], gsems.at[b]
            ).wait()

            @pl.when(s >= NWR)
            def _():
                pltpu.make_async_copy(
                    wbufs[wb].at[:, :, pl.ds(0, BBLK)],
                    out_hbm.at[s - NWR, :, wid],
                    wsems.at[wb],
                ).wait()

            transpose_chunk(gbufs[b], wbufs[wb])
            pltpu.async_copy(
                wbufs[wb].at[:, :, pl.ds(0, BBLK)],
                out_hbm.at[s, :, wid],
                wsems.at[wb],
            )

            nxt = s + NG

            @pl.when(nxt < SEQ)
            def _():
                pltpu.async_copy(
                    table_hbm.at[idx_v.at[nxt]], gbufs[b], gsems.at[b]
                )

        return carry

    lax.fori_loop(0, SEQ // NG, group_body, 0)

    for wb in range(NWR):
        s = SEQ - NWR + wb
        pltpu.make_async_copy(
            wbufs[wb].at[:, :, pl.ds(0, BBLK)], out_hbm.at[s, :, wid], wsems.at[wb]
        ).wait()


@jax.jit
def kernel(x, table):
    xt = x.T.astype(jnp.int32)
    mesh = plsc.VectorSubcoreMesh(core_axis_name="c", subcore_axis_name="s")
    run = pl.kernel(
        _embed_kernel,
        out_type=jax.ShapeDtypeStruct((SEQ, JH, NW, 8, BBLK), jnp.float32),
        mesh=mesh,
        scratch_types=[
            pltpu.VMEM((SEQ, BBLK), jnp.int32),
            [pltpu.VMEM((BBLK, D_MODEL), jnp.float32) for _ in range(NG)],
            [pltpu.VMEM((JH, 8, WPAD), jnp.float32) for _ in range(NWR)],
            pltpu.SemaphoreType.DMA((NG,)),
            pltpu.SemaphoreType.DMA((NWR,)),
        ],
        compiler_params=pltpu.CompilerParams(
            use_tc_tiling_on_sc=False, needs_layout_passes=False
        ),
    )
    p = run(xt, table)
    return p.transpose(2, 4, 0, 1, 3).reshape(BATCH, SEQ, D_MODEL)
